# Initial kernel scaffold; baseline (speedup 1.0000x reference)
import jax
import jax.numpy as jnp
from jax.experimental import pallas as pl


def kernel(cls_outs, reg_outs, images_hw):
    raise NotImplementedError("write your pallas kernel here")


# speedup vs baseline: 4.3676x; 4.3676x over previous
import functools

import jax
import jax.numpy as jnp
import numpy as np
from jax.experimental import pallas as pl
from jax.experimental.pallas import tpu as pltpu

H, W = 200, 304
STRIDE = 4
A = 3
C = 1
BATCH = 2
NUM_PRE_NMS = 2000
MAX_PER_IMG = 1000
NMS_THR = 0.7
BBOX_CLAMP = float(np.log(1000.0 / 16.0))

NPAD = 2048
ROWBLK = 256
OUTPAD = 1024


def _anchors_np():
    ratios = np.array([0.5, 1.0, 2.0], dtype=np.float64)
    scale = 8.0
    base = float(STRIDE)
    ws = (base * scale * np.sqrt(1.0 / ratios)).astype(np.float32)
    hs = (base * scale * np.sqrt(ratios)).astype(np.float32)
    base_a = np.stack([-ws / 2.0, -hs / 2.0, ws / 2.0, hs / 2.0], axis=1)

    xs = (np.arange(W, dtype=np.float32)) * STRIDE
    ys = (np.arange(H, dtype=np.float32)) * STRIDE
    sx, sy = np.meshgrid(xs, ys, indexing="xy")
    shifts = np.stack([sx.ravel(), sy.ravel(), sx.ravel(), sy.ravel()], axis=1)
    return (shifts[:, None, :] + base_a[None, :, :]).reshape(-1, 4).astype(np.float32)


def _decode_clip(ax1, ay1, ax2, ay2, dx, dy, dw, dh, wb, hb):
    aw = ax2 - ax1
    ah = ay2 - ay1
    axc = ax1 + 0.5 * aw
    ayc = ay1 + 0.5 * ah
    dw = jnp.clip(dw, -BBOX_CLAMP, BBOX_CLAMP)
    dh = jnp.clip(dh, -BBOX_CLAMP, BBOX_CLAMP)
    px = dx * aw + axc
    py = dy * ah + ayc
    pw = jnp.exp(dw) * aw
    ph = jnp.exp(dh) * ah
    x1 = jnp.clip(px - 0.5 * pw, 0.0, None)
    y1 = jnp.clip(py - 0.5 * ph, 0.0, None)
    x2 = jnp.clip(px + 0.5 * pw, 0.0, None)
    y2 = jnp.clip(py + 0.5 * ph, 0.0, None)
    x1 = jnp.minimum(x1, wb)
    y1 = jnp.minimum(y1, hb)
    x2 = jnp.minimum(x2, wb)
    y2 = jnp.minimum(y2, hb)
    return x1, y1, x2, y2


def _nms_kernel(packed_ref, packedt_ref, out_ref, sup_ref):
    P = packed_ref[0]
    PT = packedt_ref[0]

    x1, y1, x2, y2 = _decode_clip(
        P[0:1, :], P[1:2, :], P[2:3, :], P[3:4, :],
        P[4:5, :], P[5:6, :], P[6:7, :], P[7:8, :],
        P[9:10, :], P[10:11, :])
    score = P[8:9, :]
    area = (x2 - x1) * (y2 - y1)

    tx1, ty1, tx2, ty2 = _decode_clip(
        PT[:, 0:1], PT[:, 1:2], PT[:, 2:3], PT[:, 3:4],
        PT[:, 4:5], PT[:, 5:6], PT[:, 6:7], PT[:, 7:8],
        PT[:, 8:9], PT[:, 9:10])
    tarea = (tx2 - tx1) * (ty2 - ty1)

    lane_iota = jax.lax.broadcasted_iota(jnp.int32, (1, NPAD), 1)
    col_idx = lane_iota

    keep = jnp.ones((1, NPAD), dtype=jnp.float32)
    for b0 in range(NPAD // ROWBLK):
        s = b0 * ROWBLK
        rx1 = tx1[s:s + ROWBLK, :]
        ry1 = ty1[s:s + ROWBLK, :]
        rx2 = tx2[s:s + ROWBLK, :]
        ry2 = ty2[s:s + ROWBLK, :]
        rarea = tarea[s:s + ROWBLK, :]
        ltx = jnp.maximum(rx1, x1)
        lty = jnp.maximum(ry1, y1)
        rbx = jnp.minimum(rx2, x2)
        rby = jnp.minimum(ry2, y2)
        iw = jnp.clip(rbx - ltx, 0.0, None)
        ih = jnp.clip(rby - lty, 0.0, None)
        inter = iw * ih
        iou = inter / (rarea + area - inter + 1e-9)
        row_idx = s + jax.lax.broadcasted_iota(jnp.int32, (ROWBLK, 1), 0)
        sup_ref[...] = jnp.where((iou > NMS_THR) & (col_idx > row_idx), 1.0, 0.0)

        def body(i, kv):
            gi = s + i
            onehot = jnp.where(lane_iota == gi, 1.0, 0.0)
            keep_i = jnp.sum(kv * onehot)
            row = sup_ref[pl.ds(i, 1), :]
            return kv * (1.0 - keep_i * row)

        keep = jax.lax.fori_loop(0, ROWBLK, body, keep)

    real = jnp.where(score > -0.5, 1.0, 0.0)
    valid = keep * real
    incl = valid
    sh = 1
    while sh < NPAD:
        shifted = jnp.concatenate(
            [jnp.zeros((1, sh), jnp.float32), incl[:, :NPAD - sh]], axis=1)
        incl = incl + shifted
        sh *= 2
    rank = incl - valid
    sel = valid * jnp.where(rank < OUTPAD, 1.0, 0.0)

    iota_r = jax.lax.broadcasted_iota(jnp.int32, (OUTPAD, NPAD), 0)
    rank_b = jnp.broadcast_to(rank.astype(jnp.int32), (OUTPAD, NPAD))
    sel_b = jnp.broadcast_to(sel, (OUTPAD, NPAD))
    pt_mat = jnp.where(iota_r == rank_b, 1.0, 0.0) * sel_b

    ones_c = jnp.ones((NPAD, 1), jnp.float32)
    data = jnp.concatenate(
        [tx1, ty1, tx2, ty2, PT[:, 10:11], ones_c,
         jnp.zeros((NPAD, 122), jnp.float32)], axis=1)
    out_ref[0] = jax.lax.dot_general(
        pt_mat, data, (((1,), (0,)), ((), ())),
        preferred_element_type=jnp.float32)


@jax.jit
def kernel(cls_outs, reg_outs, images_hw):
    anchors = jnp.asarray(_anchors_np())

    scores_all = jax.nn.sigmoid(
        jnp.transpose(cls_outs, (0, 2, 3, 1)).reshape(BATCH, -1))
    topv, topi = jax.lax.top_k(scores_all, NUM_PRE_NMS)

    reg_rows = jnp.transpose(reg_outs, (0, 2, 3, 1)).reshape(BATCH, -1, 4)
    deltas_g = jnp.take_along_axis(reg_rows, topi[:, :, None], axis=1)
    anchors_g = anchors[topi]

    pad = NPAD - NUM_PRE_NMS
    anchors_p = jnp.pad(anchors_g, ((0, 0), (0, pad), (0, 0)))
    deltas_p = jnp.pad(deltas_g, ((0, 0), (0, pad), (0, 0)))
    scores_p = jnp.pad(topv, ((0, 0), (0, pad)), constant_values=-1.0)

    hwf = images_hw.astype(jnp.float32)
    wb = jnp.broadcast_to(hwf[:, 1][:, None, None], (BATCH, 1, NPAD))
    hb = jnp.broadcast_to(hwf[:, 0][:, None, None], (BATCH, 1, NPAD))

    packed = jnp.concatenate(
        [jnp.transpose(anchors_p, (0, 2, 1)),
         jnp.transpose(deltas_p, (0, 2, 1)),
         scores_p[:, None, :], wb, hb,
         jnp.zeros((BATCH, 5, NPAD), jnp.float32)], axis=1)

    wcol = jnp.broadcast_to(hwf[:, 1][:, None, None], (BATCH, NPAD, 1))
    hcol = jnp.broadcast_to(hwf[:, 0][:, None, None], (BATCH, NPAD, 1))
    packedt = jnp.concatenate(
        [anchors_p, deltas_p, wcol, hcol, scores_p[:, :, None],
         jnp.zeros((BATCH, NPAD, 117), jnp.float32)], axis=2)

    out = pl.pallas_call(
        _nms_kernel,
        grid=(BATCH,),
        in_specs=[
            pl.BlockSpec((1, 16, NPAD), lambda b: (b, 0, 0)),
            pl.BlockSpec((1, NPAD, 128), lambda b: (b, 0, 0)),
        ],
        out_specs=pl.BlockSpec((1, OUTPAD, 128), lambda b: (b, 0, 0)),
        out_shape=jax.ShapeDtypeStruct((BATCH, OUTPAD, 128), jnp.float32),
        scratch_shapes=[pltpu.VMEM((ROWBLK, NPAD), jnp.float32)],
    )(packed, packedt)

    res = out[:, :MAX_PER_IMG, :]
    out_boxes = res[:, :, 0:4]
    out_scores = res[:, :, 4]
    validf = res[:, :, 5]
    out_labels = jnp.where(validf > 0.5, 0, -1).astype(jnp.int32)
    return out_boxes, out_scores, out_labels
